# Initial kernel scaffold; baseline (speedup 1.0000x reference)
import jax
import jax.numpy as jnp
from jax.experimental import pallas as pl


def kernel(x_user, x_item, edge_index_clicks, edge_index_rev, W_clicks, W_rev):
    raise NotImplementedError("write your pallas kernel here")


# speedup vs baseline: 5.4467x; 5.4467x over previous
import functools

import jax
import jax.numpy as jnp
from jax import lax
from jax.experimental import pallas as pl
from jax.experimental.pallas import tpu as pltpu
from jax.experimental.pallas import tpu_sc as plsc

N = 10000
E = 320000
D = 128
NCORE = 2
NSUB = 16
CHUNK = 128
G = 8
NBLK = 20
NCHUNK = NBLK * G
E_PAD = NCHUNK * CHUNK * NSUB
NPAD = 10240
ROWS_PER_SUB = NPAD // NSUB

_mesh = plsc.VectorSubcoreMesh(
    core_axis_name="c", subcore_axis_name="s", num_cores=NCORE, num_subcores=NSUB
)


@functools.partial(
    pl.kernel,
    out_type=(
        jax.ShapeDtypeStruct((NCORE, NPAD), jnp.float32),
        jax.ShapeDtypeStruct((NCORE, NPAD), jnp.float32),
    ),
    mesh=_mesh,
    scratch_types=[
        pltpu.VMEM((NCHUNK, CHUNK), jnp.int32),
        pltpu.VMEM((NCHUNK, CHUNK), jnp.int32),
        pltpu.VMEM((CHUNK,), jnp.float32),
        pltpu.VMEM_SHARED((NPAD,), jnp.float32),
        pltpu.VMEM_SHARED((NPAD,), jnp.float32),
        pltpu.SemaphoreType.DMA,
        pltpu.SemaphoreType.DMA,
    ],
)
def _deg_kernel(src_hbm, dst_hbm, zeros_hbm, ones_hbm, dsrc_hbm, ddst_hbm,
                idx_sv, idx_dv, ones_v, deg_a, deg_b, sem_a, sem_b):
    c = lax.axis_index("c")
    s = lax.axis_index("s")
    sl = pl.ds(s * ROWS_PER_SUB, ROWS_PER_SUB)
    pltpu.sync_copy(zeros_hbm, deg_a.at[sl])
    pltpu.sync_copy(zeros_hbm, deg_b.at[sl])
    pltpu.sync_copy(ones_hbm, ones_v)
    pltpu.sync_copy(src_hbm.at[c, s], idx_sv)
    pltpu.sync_copy(dst_hbm.at[c, s], idx_dv)
    plsc.subcore_barrier()

    def body(j, carry):
        pltpu.async_copy(ones_v, deg_a.at[idx_sv.at[j]], sem_a, add=True)
        pltpu.async_copy(ones_v, deg_b.at[idx_dv.at[j]], sem_b, add=True)
        return carry

    lax.fori_loop(0, NCHUNK, body, 0)
    pltpu.make_async_copy(src_hbm.at[c, s], idx_sv, sem_a).wait()
    pltpu.make_async_copy(dst_hbm.at[c, s], idx_dv, sem_b).wait()
    plsc.subcore_barrier()
    pltpu.sync_copy(deg_a.at[sl], dsrc_hbm.at[c].at[sl])
    pltpu.sync_copy(deg_b.at[sl], ddst_hbm.at[c].at[sl])


@functools.partial(
    pl.kernel,
    out_type=jax.ShapeDtypeStruct((NCORE, NPAD, D), jnp.float32),
    mesh=_mesh,
    scratch_types=[
        pltpu.VMEM((2, G, CHUNK), jnp.int32),
        pltpu.VMEM((2, G, CHUNK), jnp.int32),
        pltpu.VMEM((2, CHUNK, D), jnp.float32),
        pltpu.VMEM_SHARED((NPAD, D), jnp.float32),
        pltpu.SemaphoreType.DMA,
        pltpu.SemaphoreType.DMA,
        pltpu.SemaphoreType.DMA,
        pltpu.SemaphoreType.DMA,
    ],
)
def _agg_kernel(tab_hbm, src_hbm, dst_hbm, zrows_hbm, agg_hbm,
                idx_sv, idx_dv, rows_v, agg_sp, isem, gsem, ssem0, ssem1):
    c = lax.axis_index("c")
    s = lax.axis_index("s")
    sl = pl.ds(s * ROWS_PER_SUB, ROWS_PER_SUB)
    pltpu.sync_copy(zrows_hbm, agg_sp.at[sl])
    pltpu.sync_copy(src_hbm.at[c, s, pl.ds(0, G)], idx_sv.at[0])
    pltpu.sync_copy(dst_hbm.at[c, s, pl.ds(0, G)], idx_dv.at[0])
    plsc.subcore_barrier()
    tab = tab_hbm.at[c]
    bufs = (rows_v.at[0], rows_v.at[1])
    ssems = (ssem0, ssem1)

    def block(k, carry):
        kb = lax.rem(k, 2)
        nb = 1 - kb

        @pl.when(k >= 1)
        def _drain_prev():
            pltpu.make_async_copy(
                bufs[0], agg_sp.at[idx_dv.at[nb, G - 2]], ssems[0]).wait()
            pltpu.make_async_copy(
                bufs[1], agg_sp.at[idx_dv.at[nb, G - 1]], ssems[1]).wait()

        @pl.when(k + 1 < NBLK)
        def _prefetch_idx():
            blk = pl.ds((k + 1) * G, G)
            pltpu.async_copy(src_hbm.at[c, s, blk], idx_sv.at[nb], isem)
            pltpu.async_copy(dst_hbm.at[c, s, blk], idx_dv.at[nb], isem)

        for g in range(G):
            b = g % 2
            if g >= 2:
                pltpu.make_async_copy(
                    bufs[b], agg_sp.at[idx_dv.at[kb, g - 2]], ssems[b]).wait()
            pltpu.async_copy(tab.at[idx_sv.at[kb, g]], bufs[b], gsem).wait()
            pltpu.async_copy(bufs[b], agg_sp.at[idx_dv.at[kb, g]],
                             ssems[b], add=True)

        @pl.when(k + 1 < NBLK)
        def _wait_idx():
            blk = pl.ds((k + 1) * G, G)
            pltpu.make_async_copy(src_hbm.at[c, s, blk], idx_sv.at[nb], isem).wait()
            pltpu.make_async_copy(dst_hbm.at[c, s, blk], idx_dv.at[nb], isem).wait()

        return carry

    lax.fori_loop(0, NBLK, block, 0)
    pltpu.make_async_copy(bufs[0], agg_sp.at[idx_dv.at[1, G - 2]], ssems[0]).wait()
    pltpu.make_async_copy(bufs[1], agg_sp.at[idx_dv.at[1, G - 1]], ssems[1]).wait()
    plsc.subcore_barrier()
    pltpu.sync_copy(agg_sp.at[sl], agg_hbm.at[c].at[sl])


def _scale_body(xu_ref, xi_ref, du_ref, di_ref, ou_ref, oi_ref):
    du = du_ref[...]
    ou_ref[...] = xu_ref[...] * jnp.where(du > 0, lax.rsqrt(du), 0.0)
    di = di_ref[...]
    oi_ref[...] = xi_ref[...] * jnp.where(di > 0, lax.rsqrt(di), 0.0)


_scale_call = pl.pallas_call(
    _scale_body,
    out_shape=(
        jax.ShapeDtypeStruct((NPAD, D), jnp.float32),
        jax.ShapeDtypeStruct((NPAD, D), jnp.float32),
    ),
)


def _out_body(ai_ref, au_ref, wc_ref, wr_ref, di_ref, du_ref, oi_ref, ou_ref):
    di = di_ref[...]
    oi_ref[...] = jnp.dot(
        ai_ref[...], wc_ref[...], preferred_element_type=jnp.float32
    ) * jnp.where(di > 0, lax.rsqrt(di), 0.0)
    du = du_ref[...]
    ou_ref[...] = jnp.dot(
        au_ref[...], wr_ref[...], preferred_element_type=jnp.float32
    ) * jnp.where(du > 0, lax.rsqrt(du), 0.0)


_out_call = pl.pallas_call(
    _out_body,
    out_shape=(
        jax.ShapeDtypeStruct((NPAD, D), jnp.float32),
        jax.ShapeDtypeStruct((NPAD, D), jnp.float32),
    ),
)


def kernel(x_user, x_item, edge_index_clicks, edge_index_rev, W_clicks, W_rev):
    ei_c = edge_index_clicks.astype(jnp.int32)
    ei_r = edge_index_rev.astype(jnp.int32)
    pad = jnp.full((E_PAD - E,), N, dtype=jnp.int32)

    def prep(a):
        return jnp.concatenate([a, pad]).reshape(NSUB, NCHUNK, CHUNK)

    src = jnp.stack([prep(ei_c[0]), prep(ei_r[0])])
    dst = jnp.stack([prep(ei_c[1]), prep(ei_r[1])])

    zeros1 = jnp.zeros((ROWS_PER_SUB,), jnp.float32)
    ones1 = jnp.ones((CHUNK,), jnp.float32)
    deg_src, deg_dst = _deg_kernel(src, dst, zeros1, ones1)

    xu_pad = jnp.concatenate([x_user, jnp.zeros((NPAD - N, D), jnp.float32)])
    xi_pad = jnp.concatenate([x_item, jnp.zeros((NPAD - N, D), jnp.float32)])
    xs_u, xs_i = _scale_call(
        xu_pad, xi_pad, deg_src[0][:, None], deg_src[1][:, None]
    )

    tab = jnp.stack([xs_u, xs_i])
    zrows = jnp.zeros((ROWS_PER_SUB, D), jnp.float32)
    agg = _agg_kernel(tab, src, dst, zrows)

    out_item, out_user = _out_call(
        agg[0], agg[1], W_clicks, W_rev, deg_dst[0][:, None], deg_dst[1][:, None]
    )
    return out_user[:N], out_item[:N]


# speedup vs baseline: 5.4919x; 1.0083x over previous
import functools

import jax
import jax.numpy as jnp
from jax import lax
from jax.experimental import pallas as pl
from jax.experimental.pallas import tpu as pltpu
from jax.experimental.pallas import tpu_sc as plsc

N = 10000
E = 320000
D = 128
NCORE = 2
NSUB = 16
CHUNK = 128
G = 16
NBLK = 10
NCHUNK = NBLK * G
E_PAD = NCHUNK * CHUNK * NSUB
NPAD = 10240
ROWS_PER_SUB = NPAD // NSUB

_mesh = plsc.VectorSubcoreMesh(
    core_axis_name="c", subcore_axis_name="s", num_cores=NCORE, num_subcores=NSUB
)


@functools.partial(
    pl.kernel,
    out_type=(
        jax.ShapeDtypeStruct((NCORE, NPAD), jnp.float32),
        jax.ShapeDtypeStruct((NCORE, NPAD), jnp.float32),
    ),
    mesh=_mesh,
    scratch_types=[
        pltpu.VMEM((NCHUNK, CHUNK), jnp.int32),
        pltpu.VMEM((NCHUNK, CHUNK), jnp.int32),
        pltpu.VMEM((CHUNK,), jnp.float32),
        pltpu.VMEM_SHARED((NPAD,), jnp.float32),
        pltpu.VMEM_SHARED((NPAD,), jnp.float32),
        pltpu.SemaphoreType.DMA,
        pltpu.SemaphoreType.DMA,
    ],
)
def _deg_kernel(src_hbm, dst_hbm, zeros_hbm, ones_hbm, dsrc_hbm, ddst_hbm,
                idx_sv, idx_dv, ones_v, deg_a, deg_b, sem_a, sem_b):
    c = lax.axis_index("c")
    s = lax.axis_index("s")
    sl = pl.ds(s * ROWS_PER_SUB, ROWS_PER_SUB)
    pltpu.sync_copy(zeros_hbm, deg_a.at[sl])
    pltpu.sync_copy(zeros_hbm, deg_b.at[sl])
    pltpu.sync_copy(ones_hbm, ones_v)
    pltpu.sync_copy(src_hbm.at[c, s], idx_sv)
    pltpu.sync_copy(dst_hbm.at[c, s], idx_dv)
    plsc.subcore_barrier()

    def body(j, carry):
        pltpu.async_copy(ones_v, deg_a.at[idx_sv.at[j]], sem_a, add=True)
        pltpu.async_copy(ones_v, deg_b.at[idx_dv.at[j]], sem_b, add=True)
        return carry

    lax.fori_loop(0, NCHUNK, body, 0)
    pltpu.make_async_copy(src_hbm.at[c, s], idx_sv, sem_a).wait()
    pltpu.make_async_copy(dst_hbm.at[c, s], idx_dv, sem_b).wait()
    plsc.subcore_barrier()
    pltpu.sync_copy(deg_a.at[sl], dsrc_hbm.at[c].at[sl])
    pltpu.sync_copy(deg_b.at[sl], ddst_hbm.at[c].at[sl])


@functools.partial(
    pl.kernel,
    out_type=jax.ShapeDtypeStruct((NCORE, NPAD, D), jnp.float32),
    mesh=_mesh,
    scratch_types=[
        pltpu.VMEM((2, G, CHUNK), jnp.int32),
        pltpu.VMEM((2, G, CHUNK), jnp.int32),
        pltpu.VMEM((2, CHUNK, D), jnp.float32),
        pltpu.VMEM_SHARED((NPAD, D), jnp.float32),
        pltpu.SemaphoreType.DMA,
        pltpu.SemaphoreType.DMA,
        pltpu.SemaphoreType.DMA,
        pltpu.SemaphoreType.DMA,
        pltpu.SemaphoreType.DMA,
    ],
)
def _agg_kernel(tab_hbm, src_hbm, dst_hbm, zrows_hbm, agg_hbm,
                idx_sv, idx_dv, rows_v, agg_sp, isem, gsem0, gsem1,
                ssem0, ssem1):
    c = lax.axis_index("c")
    s = lax.axis_index("s")
    sl = pl.ds(s * ROWS_PER_SUB, ROWS_PER_SUB)
    pltpu.sync_copy(zrows_hbm, agg_sp.at[sl])
    pltpu.sync_copy(src_hbm.at[c, s, pl.ds(0, G)], idx_sv.at[0])
    pltpu.sync_copy(dst_hbm.at[c, s, pl.ds(0, G)], idx_dv.at[0])
    plsc.subcore_barrier()
    tab = tab_hbm.at[c]
    bufs = (rows_v.at[0], rows_v.at[1])
    ssems = (ssem0, ssem1)
    gsems = (gsem0, gsem1)

    def block(k, carry):
        kb = lax.rem(k, 2)
        nb = 1 - kb

        @pl.when(k >= 1)
        def _drain_prev():
            pltpu.make_async_copy(
                bufs[0], agg_sp.at[idx_dv.at[nb, G - 2]], ssems[0]).wait()
            pltpu.make_async_copy(
                bufs[1], agg_sp.at[idx_dv.at[nb, G - 1]], ssems[1]).wait()

        @pl.when(k + 1 < NBLK)
        def _prefetch_idx():
            blk = pl.ds((k + 1) * G, G)
            pltpu.async_copy(src_hbm.at[c, s, blk], idx_sv.at[nb], isem)
            pltpu.async_copy(dst_hbm.at[c, s, blk], idx_dv.at[nb], isem)

        pltpu.async_copy(tab.at[idx_sv.at[kb, 0]], bufs[0], gsems[0])
        for g in range(G):
            b = g % 2
            nbuf = 1 - b
            pltpu.make_async_copy(tab.at[idx_sv.at[kb, g]], bufs[b],
                                  gsems[b]).wait()
            if g + 1 < G:
                if g >= 1:
                    pltpu.make_async_copy(
                        bufs[nbuf], agg_sp.at[idx_dv.at[kb, g - 1]],
                        ssems[nbuf]).wait()
                pltpu.async_copy(tab.at[idx_sv.at[kb, g + 1]], bufs[nbuf],
                                 gsems[nbuf])
            pltpu.async_copy(bufs[b], agg_sp.at[idx_dv.at[kb, g]],
                             ssems[b], add=True)

        @pl.when(k + 1 < NBLK)
        def _wait_idx():
            blk = pl.ds((k + 1) * G, G)
            pltpu.make_async_copy(src_hbm.at[c, s, blk], idx_sv.at[nb], isem).wait()
            pltpu.make_async_copy(dst_hbm.at[c, s, blk], idx_dv.at[nb], isem).wait()

        return carry

    lax.fori_loop(0, NBLK, block, 0)
    pltpu.make_async_copy(bufs[0], agg_sp.at[idx_dv.at[1, G - 2]], ssems[0]).wait()
    pltpu.make_async_copy(bufs[1], agg_sp.at[idx_dv.at[1, G - 1]], ssems[1]).wait()
    plsc.subcore_barrier()
    pltpu.sync_copy(agg_sp.at[sl], agg_hbm.at[c].at[sl])


def _scale_body(xu_ref, xi_ref, du_ref, di_ref, ou_ref, oi_ref):
    du = du_ref[...]
    ou_ref[...] = xu_ref[...] * jnp.where(du > 0, lax.rsqrt(du), 0.0)
    di = di_ref[...]
    oi_ref[...] = xi_ref[...] * jnp.where(di > 0, lax.rsqrt(di), 0.0)


_scale_call = pl.pallas_call(
    _scale_body,
    out_shape=(
        jax.ShapeDtypeStruct((NPAD, D), jnp.float32),
        jax.ShapeDtypeStruct((NPAD, D), jnp.float32),
    ),
)


def _out_body(ai_ref, au_ref, wc_ref, wr_ref, di_ref, du_ref, oi_ref, ou_ref):
    di = di_ref[...]
    oi_ref[...] = jnp.dot(
        ai_ref[...], wc_ref[...], preferred_element_type=jnp.float32
    ) * jnp.where(di > 0, lax.rsqrt(di), 0.0)
    du = du_ref[...]
    ou_ref[...] = jnp.dot(
        au_ref[...], wr_ref[...], preferred_element_type=jnp.float32
    ) * jnp.where(du > 0, lax.rsqrt(du), 0.0)


_out_call = pl.pallas_call(
    _out_body,
    out_shape=(
        jax.ShapeDtypeStruct((NPAD, D), jnp.float32),
        jax.ShapeDtypeStruct((NPAD, D), jnp.float32),
    ),
)


def kernel(x_user, x_item, edge_index_clicks, edge_index_rev, W_clicks, W_rev):
    ei_c = edge_index_clicks.astype(jnp.int32)
    ei_r = edge_index_rev.astype(jnp.int32)
    pad = jnp.full((E_PAD - E,), N, dtype=jnp.int32)

    def prep(a):
        return jnp.concatenate([a, pad]).reshape(NSUB, NCHUNK, CHUNK)

    src = jnp.stack([prep(ei_c[0]), prep(ei_r[0])])
    dst = jnp.stack([prep(ei_c[1]), prep(ei_r[1])])

    zeros1 = jnp.zeros((ROWS_PER_SUB,), jnp.float32)
    ones1 = jnp.ones((CHUNK,), jnp.float32)
    deg_src, deg_dst = _deg_kernel(src, dst, zeros1, ones1)

    xu_pad = jnp.concatenate([x_user, jnp.zeros((NPAD - N, D), jnp.float32)])
    xi_pad = jnp.concatenate([x_item, jnp.zeros((NPAD - N, D), jnp.float32)])
    xs_u, xs_i = _scale_call(
        xu_pad, xi_pad, deg_src[0][:, None], deg_src[1][:, None]
    )

    tab = jnp.stack([xs_u, xs_i])
    zrows = jnp.zeros((ROWS_PER_SUB, D), jnp.float32)
    agg = _agg_kernel(tab, src, dst, zrows)

    out_item, out_user = _out_call(
        agg[0], agg[1], W_clicks, W_rev, deg_dst[0][:, None], deg_dst[1][:, None]
    )
    return out_user[:N], out_item[:N]
